# Initial kernel scaffold; baseline (speedup 1.0000x reference)
import jax
import jax.numpy as jnp
from jax.experimental import pallas as pl


def kernel(embedded_words, action_vectors, W, b):
    raise NotImplementedError("write your pallas kernel here")


# speedup vs baseline: 12.1638x; 12.1638x over previous
import numpy as np
import jax
import jax.numpy as jnp
from jax.experimental import pallas as pl
from jax.experimental.pallas import tpu as pltpu

_ACTION_INDEX = np.array([0, 0, 0, 0, 1, 1, 1, 1, 1, 2, 3], dtype=np.int32)
_A = 4
_P = 11
_D = 128
_LANES = 128
_T = 1024

_GROUPS = []
for _a in range(_A):
    _idx = np.nonzero(_ACTION_INDEX == _a)[0]
    _GROUPS.append((int(_idx.min()), int(_idx.max()) + 1))
    assert np.all(_idx == np.arange(_idx.min(), _idx.max() + 1))


def _decoder_kernel(x_ref, av_ref, w_ref, b_ref, o_ref):
    x = x_ref[...]
    av = av_ref[...]
    num = jnp.dot(x, av, preferred_element_type=jnp.float32)
    xn = jnp.sqrt(jnp.sum(x * x, axis=1, keepdims=True))
    avn = jnp.sqrt(jnp.sum(av * av, axis=0, keepdims=True))
    denom = jnp.maximum(xn, 1e-8) * jnp.maximum(avn, 1e-8)
    sims = num / denom

    col = jax.lax.broadcasted_iota(jnp.int32, sims.shape, 1)
    neg = jnp.float32(-jnp.inf)
    pooled = []
    for lo, hi in _GROUPS:
        m = jnp.where((col >= lo) & (col < hi), sims, neg)
        pooled.append(jnp.max(m, axis=1, keepdims=True))

    logits = []
    for j in range(_A):
        acc = jnp.full_like(pooled[0], b_ref[j])
        for a in range(_A):
            acc = acc + w_ref[j, a] * pooled[a]
        logits.append(acc)

    mx = jnp.maximum(jnp.maximum(logits[0], logits[1]),
                     jnp.maximum(logits[2], logits[3]))
    exps = [jnp.exp(l - mx) for l in logits]
    ssum = exps[0] + exps[1] + exps[2] + exps[3]
    out = jnp.concatenate([e / ssum for e in exps], axis=1)
    o_ref[...] = out


def kernel(embedded_words, action_vectors, W, b):
    B = embedded_words.shape[0]
    av = action_vectors[0]
    av_pad = jnp.zeros((_D, _LANES), jnp.float32).at[:, :_P].set(av)

    grid = (B // _T,)
    out = pl.pallas_call(
        _decoder_kernel,
        grid=grid,
        in_specs=[
            pl.BlockSpec((_T, _D), lambda i: (i, 0)),
            pl.BlockSpec((_D, _LANES), lambda i: (0, 0)),
            pl.BlockSpec(memory_space=pltpu.SMEM),
            pl.BlockSpec(memory_space=pltpu.SMEM),
        ],
        out_specs=pl.BlockSpec((_T, _A), lambda i: (i, 0)),
        out_shape=jax.ShapeDtypeStruct((B, _A), jnp.float32),
        compiler_params=pltpu.CompilerParams(
            dimension_semantics=("parallel",),
        ),
    )(embedded_words, av_pad, W, b)
    return out
